# baseline (speedup 1.0000x reference)
import functools
import math

import jax
import jax.numpy as jnp
from jax import lax
from jax.experimental import pallas as pl
from jax.experimental.pallas import tpu as pltpu
from jax.experimental.pallas import tpu_sc as plsc

D_MODEL = 64
SCALE = math.sqrt(D_MODEL)

NC, NS, LANES = 2, 16, 16
NW = NC * NS
BPW = 128
TD = D_MODEL // 8
NJ = BPW // LANES


def _make_lookup(R, S, V):
    assert R == NW * BPW and S == 200
    mesh = plsc.VectorSubcoreMesh(core_axis_name="c", subcore_axis_name="s")

    @functools.partial(
        pl.kernel,
        out_type=jax.ShapeDtypeStruct((S, TD, NW, 8, BPW), jnp.float32),
        mesh=mesh,
        scratch_types=[
            pltpu.VMEM((BPW, 128), jnp.int32),
            pltpu.VMEM((BPW, 128), jnp.int32),
            pltpu.VMEM((S, BPW), jnp.int32),
            pltpu.VMEM((2, BPW, D_MODEL), jnp.float32),
            pltpu.VMEM((2, TD, 8, BPW), jnp.float32),
        ]
        + [pltpu.SemaphoreType.DMA] * 4,
        compiler_params=pltpu.CompilerParams(
            use_tc_tiling_on_sc=False, needs_layout_passes=False
        ),
    )
    def lookup(table_hbm, x1_hbm, x2_hbm, out_hbm, xa_v, xb_v, xt_v, g_v, t_v,
               g0, g1, w0, w1):
        gsem, wsem = (g0, g1), (w0, w1)
        iotas = [lax.iota(jnp.int32, LANES) + 16 * j for j in range(NJ)]
        wid = lax.axis_index("s") * NC + lax.axis_index("c")
        b0 = pl.multiple_of(wid * BPW, BPW)
        pltpu.sync_copy(x1_hbm.at[pl.ds(b0, BPW)], xa_v)
        pltpu.sync_copy(x2_hbm.at[pl.ds(b0, BPW)], xb_v)

        def xpose(src, dst_off):
            def body(s, _):
                sv = jnp.full((LANES,), s, jnp.int32)
                for j in range(NJ):
                    xt_v[s + dst_off, pl.ds(16 * j, LANES)] = (
                        plsc.load_gather(src, [iotas[j], sv])
                    )
                return ()
            return body

        lax.fori_loop(0, 128, xpose(xa_v, 0), ())
        lax.fori_loop(0, S - 128, xpose(xb_v, 128), ())

        def fire(s, par):
            pltpu.async_copy(table_hbm.at[xt_v.at[s]], g_v.at[par], gsem[par])

        def drain_gather(par):
            pltpu.make_async_copy(
                table_hbm.at[xt_v.at[0]], g_v.at[par], gsem[par]
            ).wait()

        def drain_wb(par):
            for td in range(TD):
                pltpu.make_async_copy(
                    t_v.at[par, td], out_hbm.at[0, td, 0], wsem[par]
                ).wait()

        fire(0, 0)
        fire(1, 1)

        def stage(s, par):
            drain_gather(par)

            @pl.when(s > 1)
            def _():
                drain_wb(par)

            def tbody(td, _):
                for rd in range(8):
                    d = jnp.full((LANES,), td * 8 + rd, jnp.int32)
                    for j in range(NJ):
                        t_v[par, td, rd, pl.ds(16 * j, LANES)] = (
                            plsc.load_gather(g_v.at[par], [iotas[j], d]) * SCALE
                        )
                return ()

            lax.fori_loop(0, TD, tbody, ())
            fire(jnp.minimum(s + 2, S - 1), par)
            for td in range(TD):
                pltpu.async_copy(
                    t_v.at[par, td], out_hbm.at[s, td, wid], wsem[par]
                )

        def pair(i, _):
            stage(2 * i, 0)
            stage(2 * i + 1, 1)
            return ()

        lax.fori_loop(0, S // 2, pair, ())
        for par in range(2):
            drain_gather(par)
            drain_wb(par)

    return lookup


def kernel(x, table):
    R, S = x.shape
    V = table.shape[0]
    xi = x.astype(jnp.int32)
    x1 = xi[:, :128]
    x2 = jnp.pad(xi[:, 128:], ((0, 0), (0, 256 - S)))
    out5 = _make_lookup(R, S, V)(table, x1, x2)
    return out5.transpose(2, 4, 0, 1, 3).reshape(R, S, D_MODEL)


# speedup vs baseline: 1.7561x; 1.7561x over previous
import functools
import math

import jax
import jax.numpy as jnp
from jax import lax
from jax.experimental import pallas as pl
from jax.experimental.pallas import tpu as pltpu
from jax.experimental.pallas import tpu_sc as plsc

D_MODEL = 64
SCALE = math.sqrt(D_MODEL)

NC, NS, LANES = 2, 16, 16
NW = NC * NS
BPW = 128
TD = D_MODEL // 8
PITCH = BPW + 1


def _make_lookup(R, S, V):
    assert R == NW * BPW and S == 200
    mesh = plsc.VectorSubcoreMesh(core_axis_name="c", subcore_axis_name="s")

    @functools.partial(
        pl.kernel,
        out_type=jax.ShapeDtypeStruct((S, TD, NW, 8, BPW), jnp.float32),
        mesh=mesh,
        scratch_types=[
            pltpu.VMEM((BPW, 128), jnp.int32),
            pltpu.VMEM((BPW, 128), jnp.int32),
            pltpu.VMEM((208, PITCH), jnp.int32),
            pltpu.VMEM((2, BPW, D_MODEL), jnp.float32),
            pltpu.VMEM((2, D_MODEL, PITCH), jnp.float32),
        ]
        + [pltpu.SemaphoreType.DMA] * 4,
        compiler_params=pltpu.CompilerParams(
            use_tc_tiling_on_sc=False, needs_layout_passes=False
        ),
    )
    def lookup(table_hbm, x1_hbm, x2_hbm, out_hbm, xa_v, xb_v, xt_v, g_v, t_v,
               g0, g1, w0, w1):
        gsem, wsem = (g0, g1), (w0, w1)
        iota = lax.iota(jnp.int32, LANES)
        wid = lax.axis_index("s") * NC + lax.axis_index("c")
        b0 = pl.multiple_of(wid * BPW, BPW)
        pltpu.sync_copy(x1_hbm.at[pl.ds(b0, BPW)], xa_v)
        pltpu.sync_copy(x2_hbm.at[pl.ds(b0, BPW)], xb_v)

        def xpose(src, dst_off, groups):
            svecs = [iota + 16 * j + dst_off for j in range(groups)]

            def body(cb, _):
                cbv = jnp.full((LANES,), cb, jnp.int32)
                for j in range(groups):
                    plsc.store_scatter(
                        xt_v, [svecs[j], cbv],
                        src[cb, pl.ds(16 * j, LANES)],
                    )
                return ()
            return body

        lax.fori_loop(0, BPW, xpose(xa_v, 0, 8), ())
        lax.fori_loop(0, BPW, xpose(xb_v, 128, (S - 128) // LANES + 1), ())

        def fire(s, par):
            pltpu.async_copy(
                table_hbm.at[xt_v.at[s].at[pl.ds(0, BPW)]], g_v.at[par],
                gsem[par],
            )

        def drain_gather(par):
            pltpu.make_async_copy(
                table_hbm.at[xt_v.at[0].at[pl.ds(0, BPW)]], g_v.at[par],
                gsem[par],
            ).wait()

        def drain_wb(par):
            for td in range(TD):
                pltpu.make_async_copy(
                    t_v.at[par].at[pl.ds(8 * td, 8), pl.ds(0, BPW)],
                    out_hbm.at[0, td, 0],
                    wsem[par],
                ).wait()

        fire(0, 0)
        fire(1, 1)

        dvecs = [iota + 16 * c for c in range(D_MODEL // LANES)]

        def stage(s, par):
            drain_gather(par)

            @pl.when(s > 1)
            def _():
                drain_wb(par)

            def tbody(cb, _):
                cbv = jnp.full((LANES,), cb, jnp.int32)
                for c in range(D_MODEL // LANES):
                    plsc.store_scatter(
                        t_v.at[par], [dvecs[c], cbv],
                        g_v[par, cb, pl.ds(16 * c, LANES)] * SCALE,
                    )
                return ()

            lax.fori_loop(0, BPW, tbody, ())
            fire(jnp.minimum(s + 2, S - 1), par)
            for td in range(TD):
                pltpu.async_copy(
                    t_v.at[par].at[pl.ds(8 * td, 8), pl.ds(0, BPW)],
                    out_hbm.at[s, td, wid],
                    wsem[par],
                )

        def pair(i, _):
            stage(2 * i, 0)
            stage(2 * i + 1, 1)
            return ()

        lax.fori_loop(0, S // 2, pair, ())
        for par in range(2):
            drain_gather(par)
            drain_wb(par)

    return lookup


def kernel(x, table):
    R, S = x.shape
    V = table.shape[0]
    xi = x.astype(jnp.int32)
    x1 = xi[:, :128]
    x2 = jnp.pad(xi[:, 128:], ((0, 0), (0, 256 - S)))
    out5 = _make_lookup(R, S, V)(table, x1, x2)
    return out5.transpose(2, 4, 0, 1, 3).reshape(R, S, D_MODEL)


# speedup vs baseline: 1.7870x; 1.0176x over previous
import functools
import math

import jax
import jax.numpy as jnp
from jax import lax
from jax.experimental import pallas as pl
from jax.experimental.pallas import tpu as pltpu
from jax.experimental.pallas import tpu_sc as plsc

D_MODEL = 64
SCALE = math.sqrt(D_MODEL)

NC, NS, LANES = 2, 16, 16
NW = NC * NS
BPW = 128
TD = D_MODEL // 8
PITCH = BPW + 1


def _make_lookup(R, S, V):
    assert R == NW * BPW and S == 200
    mesh = plsc.VectorSubcoreMesh(core_axis_name="c", subcore_axis_name="s")

    @functools.partial(
        pl.kernel,
        out_type=jax.ShapeDtypeStruct((S, TD, NW, 8, BPW), jnp.float32),
        mesh=mesh,
        scratch_types=[
            pltpu.VMEM((BPW, 128), jnp.int32),
            pltpu.VMEM((BPW, 128), jnp.int32),
            pltpu.VMEM((208, PITCH), jnp.int32),
            pltpu.VMEM((2, BPW, D_MODEL), jnp.float32),
            pltpu.VMEM((2, D_MODEL, PITCH), jnp.float32),
        ]
        + [pltpu.SemaphoreType.DMA] * 4,
        compiler_params=pltpu.CompilerParams(
            use_tc_tiling_on_sc=False, needs_layout_passes=False
        ),
    )
    def lookup(table_hbm, x1_hbm, x2_hbm, out_hbm, xa_v, xb_v, xt_v, g_v, t_v,
               g0, g1, w0, w1):
        gsem, wsem = (g0, g1), (w0, w1)
        iota = lax.iota(jnp.int32, LANES)
        wid = lax.axis_index("s") * NC + lax.axis_index("c")
        b0 = pl.multiple_of(wid * BPW, BPW)
        pltpu.sync_copy(x1_hbm.at[pl.ds(b0, BPW)], xa_v)
        pltpu.sync_copy(x2_hbm.at[pl.ds(b0, BPW)], xb_v)

        def xpose(src, dst_off, groups):
            svecs = [iota + 16 * j + dst_off for j in range(groups)]

            def body(cb, _):
                cbv = jnp.full((LANES,), cb, jnp.int32)
                for j in range(groups):
                    plsc.store_scatter(
                        xt_v, [svecs[j], cbv],
                        src[cb, pl.ds(16 * j, LANES)],
                    )
                return ()
            return body

        lax.fori_loop(0, BPW, xpose(xa_v, 0, 8), ())
        lax.fori_loop(0, BPW, xpose(xb_v, 128, (S - 128) // LANES + 1), ())

        def fire(s, par):
            pltpu.async_copy(
                table_hbm.at[xt_v.at[s].at[pl.ds(0, BPW)]], g_v.at[par],
                gsem[par],
            )

        def drain_gather(par):
            pltpu.make_async_copy(
                table_hbm.at[xt_v.at[0].at[pl.ds(0, BPW)]], g_v.at[par],
                gsem[par],
            ).wait()

        def drain_wb(par):
            for td in range(TD):
                pltpu.make_async_copy(
                    t_v.at[par].at[pl.ds(8 * td, 8), pl.ds(0, BPW)],
                    out_hbm.at[0, td, 0],
                    wsem[par],
                ).wait()

        fire(0, 0)
        fire(1, 1)

        dvecs = [iota + 16 * c for c in range(D_MODEL // LANES)]

        def stage(s, par):
            drain_gather(par)

            @pl.when(s > 1)
            def _():
                drain_wb(par)

            def tbody(q, _):
                for u in range(4):
                    cb = 4 * q + u
                    cbv = jnp.full((LANES,), cb, jnp.int32)
                    for c in range(D_MODEL // LANES):
                        plsc.store_scatter(
                            t_v.at[par], [dvecs[c], cbv],
                            g_v[par, cb, pl.ds(16 * c, LANES)] * SCALE,
                        )
                return ()

            lax.fori_loop(0, BPW // 4, tbody, ())
            fire(jnp.minimum(s + 2, S - 1), par)
            for td in range(TD):
                pltpu.async_copy(
                    t_v.at[par].at[pl.ds(8 * td, 8), pl.ds(0, BPW)],
                    out_hbm.at[s, td, wid],
                    wsem[par],
                )

        def pair(i, _):
            stage(2 * i, 0)
            stage(2 * i + 1, 1)
            return ()

        lax.fori_loop(0, S // 2, pair, ())
        for par in range(2):
            drain_gather(par)
            drain_wb(par)

    return lookup


def kernel(x, table):
    R, S = x.shape
    V = table.shape[0]
    xi = x.astype(jnp.int32)
    x1 = xi[:, :128]
    x2 = jnp.pad(xi[:, 128:], ((0, 0), (0, 256 - S)))
    out5 = _make_lookup(R, S, V)(table, x1, x2)
    return out5.transpose(2, 4, 0, 1, 3).reshape(R, S, D_MODEL)


# speedup vs baseline: 2.5998x; 1.4549x over previous
import functools
import math

import jax
import jax.numpy as jnp
from jax import lax
from jax.experimental import pallas as pl
from jax.experimental.pallas import tpu as pltpu
from jax.experimental.pallas import tpu_sc as plsc

D_MODEL = 64
SCALE = math.sqrt(D_MODEL)

NC, NS, LANES = 2, 16, 16
NW = NC * NS
BPW = 128
TD = D_MODEL // 8
PITCH = BPW + 1


def _make_lookup(R, S, V):
    assert R == NW * BPW and S == 200
    mesh = plsc.VectorSubcoreMesh(core_axis_name="c", subcore_axis_name="s")

    @functools.partial(
        pl.kernel,
        out_type=jax.ShapeDtypeStruct((S, TD, NW, 8, BPW), jnp.float32),
        mesh=mesh,
        scratch_types=[
            pltpu.VMEM((BPW, 128), jnp.int32),
            pltpu.VMEM((BPW, 128), jnp.int32),
            pltpu.VMEM((208, PITCH), jnp.int32),
            pltpu.VMEM((2, BPW, D_MODEL), jnp.float32),
            pltpu.VMEM((2, D_MODEL, PITCH), jnp.float32),
        ]
        + [pltpu.SemaphoreType.DMA] * 4,
        compiler_params=pltpu.CompilerParams(
            use_tc_tiling_on_sc=False, needs_layout_passes=False
        ),
    )
    def lookup(table_hbm, x1_hbm, x2_hbm, out_hbm, xa_v, xb_v, xt_v, g_v, t_v,
               g0, g1, w0, w1):
        gsem, wsem = (g0, g1), (w0, w1)
        iota = lax.iota(jnp.int32, LANES)
        wid = lax.axis_index("s") * NC + lax.axis_index("c")
        b0 = pl.multiple_of(wid * BPW, BPW)
        pltpu.sync_copy(x1_hbm.at[pl.ds(b0, BPW)], xa_v)
        pltpu.sync_copy(x2_hbm.at[pl.ds(b0, BPW)], xb_v)

        def xpose(src, dst_off, groups):
            svecs = [iota + 16 * j + dst_off for j in range(groups)]

            def body(cb, _):
                cbv = jnp.full((LANES,), cb, jnp.int32)
                for j in range(groups):
                    plsc.store_scatter(
                        xt_v, [svecs[j], cbv],
                        src[cb, pl.ds(16 * j, LANES)],
                    )
                return ()
            return body

        lax.fori_loop(0, BPW, xpose(xa_v, 0, 8), ())
        lax.fori_loop(0, BPW, xpose(xb_v, 128, (S - 128) // LANES + 1), ())

        def fire(s, par):
            pltpu.async_copy(
                table_hbm.at[xt_v.at[s].at[pl.ds(0, BPW)]], g_v.at[par],
                gsem[par],
            )

        def drain_gather(par):
            pltpu.make_async_copy(
                table_hbm.at[xt_v.at[0].at[pl.ds(0, BPW)]], g_v.at[par],
                gsem[par],
            ).wait()

        def drain_wb(par):
            for td in range(TD):
                pltpu.make_async_copy(
                    t_v.at[par].at[pl.ds(8 * td, 8), pl.ds(0, BPW)],
                    out_hbm.at[0, td, 0],
                    wsem[par],
                ).wait()

        fire(0, 0)
        fire(1, 1)

        dvecs = [iota + 16 * c for c in range(D_MODEL // LANES)]

        def stage(s, par):
            drain_gather(par)

            @pl.when(s > 1)
            def _():
                drain_wb(par)

            @functools.partial(plsc.parallel_loop, 0, BPW, unroll=4)
            def _(cb):
                cbv = jnp.full((LANES,), cb, jnp.int32)
                for c in range(D_MODEL // LANES):
                    plsc.store_scatter(
                        t_v.at[par], [dvecs[c], cbv],
                        g_v[par, cb, pl.ds(16 * c, LANES)] * SCALE,
                    )
            fire(jnp.minimum(s + 2, S - 1), par)
            for td in range(TD):
                pltpu.async_copy(
                    t_v.at[par].at[pl.ds(8 * td, 8), pl.ds(0, BPW)],
                    out_hbm.at[s, td, wid],
                    wsem[par],
                )

        def pair(i, _):
            stage(2 * i, 0)
            stage(2 * i + 1, 1)
            return ()

        lax.fori_loop(0, S // 2, pair, ())
        for par in range(2):
            drain_gather(par)
            drain_wb(par)

    return lookup


def kernel(x, table):
    R, S = x.shape
    V = table.shape[0]
    xi = x.astype(jnp.int32)
    x1 = xi[:, :128]
    x2 = jnp.pad(xi[:, 128:], ((0, 0), (0, 256 - S)))
    out5 = _make_lookup(R, S, V)(table, x1, x2)
    return out5.transpose(2, 4, 0, 1, 3).reshape(R, S, D_MODEL)
